# baseline (speedup 1.0000x reference)
import functools

import jax
import jax.numpy as jnp
from jax import lax
from jax.experimental import pallas as pl
from jax.experimental.pallas import tpu as pltpu
from jax.experimental.pallas import tpu_sc as plsc


def _sc_gather_all(enc_emb, emb_mu, ls_flat, ctx_t, neg_t, cen, bb_tc):
    v, d = enc_emb.shape
    bw = ctx_t.shape[0]
    bn = cen.shape[0]
    info = plsc.get_sparse_core_info()
    nw = info.num_cores * info.num_subcores
    ch = 128
    per_w = bw // nw
    n_ch = per_w // ch
    n_pair = n_ch // 2
    cper = bn // nw
    assert per_w % ch == 0 and n_ch % 2 == 0 and bn % nw == 0
    wn_ids = bw // bn
    rpb = bb_tc // ch
    assert bb_tc % ch == 0 and bn % bb_tc == 0 and bn % ch == 0

    ctx2 = ctx_t.reshape(nw, n_ch, ch)
    neg2 = neg_t.reshape(nw, n_ch, ch)
    cen2 = cen.reshape(nw, 1, cper)
    mesh = plsc.VectorSubcoreMesh(core_axis_name="c", subcore_axis_name="s")
    f32 = jnp.float32

    @functools.partial(
        pl.kernel,
        mesh=mesh,
        compiler_params=pltpu.CompilerParams(use_tc_tiling_on_sc=False),
        out_type=[
            jax.ShapeDtypeStruct((bw // ch, ch, d), f32),
            jax.ShapeDtypeStruct((bw // ch, ch, d), f32),
            jax.ShapeDtypeStruct((bw // ch, ch, d), f32),
            jax.ShapeDtypeStruct((nw, cper, d), f32),
            jax.ShapeDtypeStruct((nw, cper, d), f32),
            jax.ShapeDtypeStruct((nw, n_ch, ch), f32),
            jax.ShapeDtypeStruct((nw, n_ch, ch), f32),
            jax.ShapeDtypeStruct((nw, 1, cper), f32),
        ],
        scratch_types=[
            pltpu.VMEM((n_ch, ch), jnp.int32),
            pltpu.VMEM((n_ch, ch), jnp.int32),
            pltpu.VMEM((1, cper), jnp.int32),
            pltpu.VMEM((ch, d), f32),
            pltpu.VMEM((ch, d), f32),
            pltpu.VMEM((cper, d), f32),
            pltpu.VMEM((cper, d), f32),
            pltpu.VMEM((n_ch, ch), f32),
            pltpu.VMEM((n_ch, ch), f32),
            pltpu.VMEM((1, cper), f32),
            pltpu.SemaphoreType.DMA,
            pltpu.SemaphoreType.DMA,
            pltpu.SemaphoreType.DMA,
        ],
    )
    def sc_kernel(enc_hbm, mu_hbm, ls_hbm, ctx_hbm, neg_hbm, cen_hbm,
                  cte_o, mupos_o, muneg_o, ce_o, muc_o, lsp_o, lsn_o, lsc_o,
                  ctxi, negi, ceni, buf0, buf1, cbuf0, cbuf1, lspb, lsnb,
                  lscb, sem0, sem1, sem2):
        wid = lax.axis_index("s") * info.num_cores + lax.axis_index("c")

        pltpu.sync_copy(ctx_hbm.at[wid], ctxi)
        pltpu.sync_copy(neg_hbm.at[wid], negi)
        pltpu.sync_copy(cen_hbm.at[wid], ceni)

        @pl.loop(0, n_ch)
        def _ls_issue(c):
            pltpu.async_copy(ls_hbm.at[ctxi.at[c]], lspb.at[c], sem2)
            pltpu.async_copy(ls_hbm.at[negi.at[c]], lsnb.at[c], sem2)

        pltpu.async_copy(ls_hbm.at[ceni.at[0]], lscb.at[0], sem2)

        base = wid * per_w

        def dst(c):
            p = base + c * ch
            j = p // bn
            i = p - j * bn
            blk = i // bb_tc
            r = (i - blk * bb_tc) // ch
            return (blk * wn_ids + j) * rpb + r

        def row_job(tbl, idxs, out):
            pltpu.async_copy(tbl.at[idxs.at[0]], buf0, sem0)

            @pl.loop(0, n_pair)
            def _pair(g):
                c0 = 2 * g
                pltpu.async_copy(tbl.at[idxs.at[c0 + 1]], buf1, sem1)
                pltpu.make_async_copy(tbl.at[idxs.at[c0]], buf0, sem0).wait()
                pltpu.sync_copy(buf0, out.at[dst(c0)])

                @pl.when(g < n_pair - 1)
                def _():
                    pltpu.async_copy(tbl.at[idxs.at[c0 + 2]], buf0, sem0)

                pltpu.make_async_copy(tbl.at[idxs.at[c0 + 1]], buf1, sem1).wait()
                pltpu.sync_copy(buf1, out.at[dst(c0 + 1)])

        row_job(enc_hbm, ctxi, cte_o)
        row_job(mu_hbm, ctxi, mupos_o)
        row_job(mu_hbm, negi, muneg_o)

        pltpu.async_copy(enc_hbm.at[ceni.at[0]], cbuf0, sem0)
        pltpu.async_copy(mu_hbm.at[ceni.at[0]], cbuf1, sem1)
        pltpu.make_async_copy(enc_hbm.at[ceni.at[0]], cbuf0, sem0).wait()
        pltpu.sync_copy(cbuf0, ce_o.at[wid])
        pltpu.make_async_copy(mu_hbm.at[ceni.at[0]], cbuf1, sem1).wait()
        pltpu.sync_copy(cbuf1, muc_o.at[wid])

        @pl.loop(0, n_ch)
        def _ls_wait(c):
            pltpu.make_async_copy(ls_hbm.at[ctxi.at[c]], lspb.at[c], sem2).wait()
            pltpu.make_async_copy(ls_hbm.at[negi.at[c]], lsnb.at[c], sem2).wait()

        pltpu.make_async_copy(ls_hbm.at[ceni.at[0]], lscb.at[0], sem2).wait()
        pltpu.sync_copy(lspb, lsp_o.at[wid])
        pltpu.sync_copy(lsnb, lsn_o.at[wid])
        pltpu.sync_copy(lscb, lsc_o.at[wid])

    return sc_kernel(enc_emb, emb_mu, ls_flat, ctx2, neg2, cen2)


def _tc_loss(ce, muc, lsc, ncnt, cte3, mupos3, muneg3, lsp, lsn,
             f_w1, f_w2, f_b, u_w, u_b, v_w, v_b, margin, total_b):
    bn, d = ce.shape
    wn = cte3.shape[1]
    h = f_w1.shape[0]
    bb = cte3.shape[2]
    nblk = bn // bb
    f32 = jnp.float32
    dn = (((1,), (1,)), ((), ()))

    dn2 = (((1,), (0,)), ((), ()))

    def body(ce_r, muc_r, lsc_r, nc_r, ncw_r, cte_r, mupos_r, muneg_r,
             lsp_r, lsn_r, fw1_r, fw2_r, fb_r, uw_r, ub_r, vw_r, vb_r,
             kl_r, mm_r):
        ib = pl.program_id(0)
        a = lax.dot_general(ce_r[...], fw1_r[...], dn, preferred_element_type=f32)
        fb_v = fb_r[...]
        nc_v = nc_r[...]
        iota = lax.broadcasted_iota(jnp.int32, (bb, wn), 1)
        maskf = (iota < nc_v).astype(f32)
        iota_j = lax.broadcasted_iota(jnp.int32, (wn, bb), 0)
        maskw = (iota_j < ncw_r[...]).astype(f32)

        cte_f = cte_r[...].reshape(wn * bb, d)
        mupos_v = mupos_r[...].reshape(wn, bb, d)
        muneg_v = muneg_r[...].reshape(wn, bb, d)
        bj = lax.dot_general(cte_f, fw2_r[...], dn,
                             preferred_element_type=f32).reshape(wn, bb, h)
        hj = jnp.maximum(bj + a[None] + fb_v[None], 0.0)
        h_sum = jnp.sum(hj * maskw[:, :, None], axis=0)
        nvalid = jnp.sum(maskf, axis=1, keepdims=True)
        h_sum = h_sum + (wn - nvalid) * jnp.maximum(fb_v, 0.0)

        mu_q = lax.dot_general(h_sum, uw_r[...], dn, preferred_element_type=f32) + ub_r[...]
        lq = (jnp.sum(h_sum * vw_r[...], axis=1, keepdims=True)
              + vb_r[...])
        var_q = jnp.exp(2.0 * lq)
        dvq = d * var_q

        lsc_v = lsc_r[...]
        dmc = mu_q - muc_r[...]
        sq_c = jnp.sum(dmc * dmc, axis=1, keepdims=True)
        klc = 0.5 * (2.0 * d * (lsc_v - lq)
                     + (dvq + sq_c) * jnp.exp(-2.0 * lsc_v) - d)
        kl_blk = jnp.sum(klc)

        dp3 = mu_q[None] - mupos_v
        dpp = dp3 * dp3
        dq3 = mu_q[None] - muneg_v
        dqq = dq3 * dq3
        sqp = jnp.zeros((bb, wn), f32)
        sqn = jnp.zeros((bb, wn), f32)
        for j in range(wn):
            cj = (lax.broadcasted_iota(jnp.int32, (d, wn), 1) == j).astype(f32)
            sqp = sqp + lax.dot_general(dpp[j], cj, dn2, preferred_element_type=f32)
            sqn = sqn + lax.dot_general(dqq[j], cj, dn2, preferred_element_type=f32)
        lsp_v = lsp_r[...]
        lsn_v = lsn_r[...]
        delta = (d * (lsp_v - lsn_v)
                 + 0.5 * ((dvq + sqp) * jnp.exp(-2.0 * lsp_v)
                          - (dvq + sqn) * jnp.exp(-2.0 * lsn_v)))
        mm_blk = jnp.sum(maskf * jnp.maximum(delta + margin, 0.0))

        @pl.when(ib == 0)
        def _():
            kl_r[...] = jnp.zeros((1, 1), f32)
            mm_r[...] = jnp.zeros((1, 1), f32)

        kl_r[...] += (kl_blk / total_b)[None, None]
        mm_r[...] += (mm_blk / total_b)[None, None]

    full = lambda b: (0, 0)
    out = pl.pallas_call(
        body,
        grid=(nblk,),
        in_specs=[
            pl.BlockSpec((bb, d), lambda b: (b, 0)),
            pl.BlockSpec((bb, d), lambda b: (b, 0)),
            pl.BlockSpec((bb, 1), lambda b: (b, 0)),
            pl.BlockSpec((bb, 1), lambda b: (b, 0)),
            pl.BlockSpec((1, bb), lambda b: (0, b)),
            pl.BlockSpec((1, wn, bb, d), lambda b: (b, 0, 0, 0)),
            pl.BlockSpec((1, wn, bb, d), lambda b: (b, 0, 0, 0)),
            pl.BlockSpec((1, wn, bb, d), lambda b: (b, 0, 0, 0)),
            pl.BlockSpec((bb, wn), lambda b: (b, 0)),
            pl.BlockSpec((bb, wn), lambda b: (b, 0)),
            pl.BlockSpec((h, d), full),
            pl.BlockSpec((h, d), full),
            pl.BlockSpec((1, h), full),
            pl.BlockSpec((d, h), full),
            pl.BlockSpec((1, d), full),
            pl.BlockSpec((1, h), full),
            pl.BlockSpec((1, 1), full),
        ],
        out_specs=[pl.BlockSpec((1, 1), full), pl.BlockSpec((1, 1), full)],
        out_shape=[jax.ShapeDtypeStruct((1, 1), f32),
                   jax.ShapeDtypeStruct((1, 1), f32)],
    )(ce, muc, lsc, ncnt, ncnt.reshape(1, bn), cte3, mupos3, muneg3, lsp, lsn,
      f_w1, f_w2, f_b, u_w, u_b, v_w, v_b)
    return out


def kernel(emb_mu, emb_log_sigma, enc_emb, f_W, f_b, u_W, u_b, v_W, v_b,
           center_ids, context_ids, neg_context_ids, num_contexts):
    bn, wn = context_ids.shape
    v, d = emb_mu.shape
    h = f_W.shape[0]
    margin = 1.0
    ls_flat = emb_log_sigma.reshape(-1)
    ncnt_all = num_contexts.reshape(bn, 1).astype(jnp.int32)

    nsplit = 1
    bs = bn // nsplit
    kl_tot = jnp.zeros((), jnp.float32)
    mm_tot = jnp.zeros((), jnp.float32)
    for s in range(nsplit):
        sl = slice(s * bs, (s + 1) * bs)
        ctx_t = context_ids[sl].T.reshape(-1).astype(jnp.int32)
        neg_t = neg_context_ids[sl].T.reshape(-1).astype(jnp.int32)
        cen = center_ids[sl].astype(jnp.int32)

        bb = 128
        (cte, mupos, muneg, ce, muc, lsp_r, lsn_r, lsc) = _sc_gather_all(
            enc_emb, emb_mu, ls_flat, ctx_t, neg_t, cen, bb)

        kl, mm = _tc_loss(
            ce.reshape(bs, d), muc.reshape(bs, d), lsc.reshape(bs, 1),
            ncnt_all[sl],
            cte.reshape(bs // bb, wn, bb, d), mupos.reshape(bs // bb, wn, bb, d),
            muneg.reshape(bs // bb, wn, bb, d),
            lsp_r.reshape(wn, bs).T, lsn_r.reshape(wn, bs).T,
            f_W[:, :d], f_W[:, d:], f_b.reshape(1, h),
            u_W, u_b.reshape(1, d), v_W, v_b.reshape(1, 1), margin, bn)
        kl_tot = kl_tot + kl[0, 0]
        mm_tot = mm_tot + mm[0, 0]
    return (kl_tot, mm_tot)


# speedup vs baseline: 1.0733x; 1.0733x over previous
import functools

import jax
import jax.numpy as jnp
from jax import lax
from jax.experimental import pallas as pl
from jax.experimental.pallas import tpu as pltpu
from jax.experimental.pallas import tpu_sc as plsc


def _sc_gather_all(enc_emb, emb_mu, ls_flat, ctx_t, neg_t, cen, bb_tc):
    v, d = enc_emb.shape
    bw = ctx_t.shape[0]
    bn = cen.shape[0]
    info = plsc.get_sparse_core_info()
    nw = info.num_cores * info.num_subcores
    ch = 256
    per_w = bw // nw
    n_ch = per_w // ch
    n_pair = n_ch // 2
    cper = bn // nw
    assert per_w % ch == 0 and n_ch % 2 == 0 and bn % nw == 0
    wn_ids = bw // bn
    rpb = bb_tc // ch
    assert bb_tc % ch == 0 and bn % bb_tc == 0 and bn % ch == 0

    ctx2 = ctx_t.reshape(nw, n_ch, ch)
    neg2 = neg_t.reshape(nw, n_ch, ch)
    cen2 = cen.reshape(nw, 1, cper)
    mesh = plsc.VectorSubcoreMesh(core_axis_name="c", subcore_axis_name="s")
    f32 = jnp.float32

    @functools.partial(
        pl.kernel,
        mesh=mesh,
        compiler_params=pltpu.CompilerParams(use_tc_tiling_on_sc=False),
        out_type=[
            jax.ShapeDtypeStruct((bw // ch, ch, d), f32),
            jax.ShapeDtypeStruct((bw // ch, ch, d), f32),
            jax.ShapeDtypeStruct((bw // ch, ch, d), f32),
            jax.ShapeDtypeStruct((nw, cper, d), f32),
            jax.ShapeDtypeStruct((nw, cper, d), f32),
            jax.ShapeDtypeStruct((nw, n_ch, ch), f32),
            jax.ShapeDtypeStruct((nw, n_ch, ch), f32),
            jax.ShapeDtypeStruct((nw, 1, cper), f32),
        ],
        scratch_types=[
            pltpu.VMEM((n_ch, ch), jnp.int32),
            pltpu.VMEM((n_ch, ch), jnp.int32),
            pltpu.VMEM((1, cper), jnp.int32),
            pltpu.VMEM((ch, d), f32),
            pltpu.VMEM((ch, d), f32),
            pltpu.VMEM((cper, d), f32),
            pltpu.VMEM((cper, d), f32),
            pltpu.VMEM((n_ch, ch), f32),
            pltpu.VMEM((n_ch, ch), f32),
            pltpu.VMEM((1, cper), f32),
            pltpu.SemaphoreType.DMA,
            pltpu.SemaphoreType.DMA,
            pltpu.SemaphoreType.DMA,
        ],
    )
    def sc_kernel(enc_hbm, mu_hbm, ls_hbm, ctx_hbm, neg_hbm, cen_hbm,
                  cte_o, mupos_o, muneg_o, ce_o, muc_o, lsp_o, lsn_o, lsc_o,
                  ctxi, negi, ceni, buf0, buf1, cbuf0, cbuf1, lspb, lsnb,
                  lscb, sem0, sem1, sem2):
        wid = lax.axis_index("s") * info.num_cores + lax.axis_index("c")

        pltpu.sync_copy(ctx_hbm.at[wid], ctxi)
        pltpu.sync_copy(neg_hbm.at[wid], negi)
        pltpu.sync_copy(cen_hbm.at[wid], ceni)

        @pl.loop(0, n_ch)
        def _ls_issue(c):
            pltpu.async_copy(ls_hbm.at[ctxi.at[c]], lspb.at[c], sem2)
            pltpu.async_copy(ls_hbm.at[negi.at[c]], lsnb.at[c], sem2)

        pltpu.async_copy(ls_hbm.at[ceni.at[0]], lscb.at[0], sem2)

        base = wid * per_w

        def dst(c):
            p = base + c * ch
            j = p // bn
            i = p - j * bn
            blk = i // bb_tc
            r = (i - blk * bb_tc) // ch
            return (blk * wn_ids + j) * rpb + r

        def row_job(tbl, idxs, out):
            pltpu.async_copy(tbl.at[idxs.at[0]], buf0, sem0)

            @pl.loop(0, n_pair)
            def _pair(g):
                c0 = 2 * g
                pltpu.async_copy(tbl.at[idxs.at[c0 + 1]], buf1, sem1)
                pltpu.make_async_copy(tbl.at[idxs.at[c0]], buf0, sem0).wait()
                pltpu.sync_copy(buf0, out.at[dst(c0)])

                @pl.when(g < n_pair - 1)
                def _():
                    pltpu.async_copy(tbl.at[idxs.at[c0 + 2]], buf0, sem0)

                pltpu.make_async_copy(tbl.at[idxs.at[c0 + 1]], buf1, sem1).wait()
                pltpu.sync_copy(buf1, out.at[dst(c0 + 1)])

        row_job(enc_hbm, ctxi, cte_o)
        row_job(mu_hbm, ctxi, mupos_o)
        row_job(mu_hbm, negi, muneg_o)

        pltpu.async_copy(enc_hbm.at[ceni.at[0]], cbuf0, sem0)
        pltpu.async_copy(mu_hbm.at[ceni.at[0]], cbuf1, sem1)
        pltpu.make_async_copy(enc_hbm.at[ceni.at[0]], cbuf0, sem0).wait()
        pltpu.sync_copy(cbuf0, ce_o.at[wid])
        pltpu.make_async_copy(mu_hbm.at[ceni.at[0]], cbuf1, sem1).wait()
        pltpu.sync_copy(cbuf1, muc_o.at[wid])

        @pl.loop(0, n_ch)
        def _ls_wait(c):
            pltpu.make_async_copy(ls_hbm.at[ctxi.at[c]], lspb.at[c], sem2).wait()
            pltpu.make_async_copy(ls_hbm.at[negi.at[c]], lsnb.at[c], sem2).wait()

        pltpu.make_async_copy(ls_hbm.at[ceni.at[0]], lscb.at[0], sem2).wait()
        pltpu.sync_copy(lspb, lsp_o.at[wid])
        pltpu.sync_copy(lsnb, lsn_o.at[wid])
        pltpu.sync_copy(lscb, lsc_o.at[wid])

    return sc_kernel(enc_emb, emb_mu, ls_flat, ctx2, neg2, cen2)


def _tc_loss(ce, muc, lsc, ncnt, cte3, mupos3, muneg3, lsp, lsn,
             f_w1, f_w2, f_b, u_w, u_b, v_w, v_b, margin, total_b):
    bn, d = ce.shape
    wn = cte3.shape[1]
    h = f_w1.shape[0]
    bb = cte3.shape[2]
    nblk = bn // bb
    f32 = jnp.float32
    dn = (((1,), (1,)), ((), ()))

    dn2 = (((1,), (0,)), ((), ()))

    def body(ce_r, muc_r, lsc_r, nc_r, ncw_r, cte_r, mupos_r, muneg_r,
             lsp_r, lsn_r, fw1_r, fw2_r, fb_r, uw_r, ub_r, vw_r, vb_r,
             kl_r, mm_r):
        ib = pl.program_id(0)
        a = lax.dot_general(ce_r[...], fw1_r[...], dn, preferred_element_type=f32)
        fb_v = fb_r[...]
        nc_v = nc_r[...]
        iota = lax.broadcasted_iota(jnp.int32, (bb, wn), 1)
        maskf = (iota < nc_v).astype(f32)
        iota_j = lax.broadcasted_iota(jnp.int32, (wn, bb), 0)
        maskw = (iota_j < ncw_r[...]).astype(f32)

        cte_f = cte_r[...].reshape(wn * bb, d)
        mupos_v = mupos_r[...].reshape(wn, bb, d)
        muneg_v = muneg_r[...].reshape(wn, bb, d)
        bj = lax.dot_general(cte_f, fw2_r[...], dn,
                             preferred_element_type=f32).reshape(wn, bb, h)
        hj = jnp.maximum(bj + a[None] + fb_v[None], 0.0)
        h_sum = jnp.sum(hj * maskw[:, :, None], axis=0)
        nvalid = jnp.sum(maskf, axis=1, keepdims=True)
        h_sum = h_sum + (wn - nvalid) * jnp.maximum(fb_v, 0.0)

        mu_q = lax.dot_general(h_sum, uw_r[...], dn, preferred_element_type=f32) + ub_r[...]
        lq = (jnp.sum(h_sum * vw_r[...], axis=1, keepdims=True)
              + vb_r[...])
        var_q = jnp.exp(2.0 * lq)
        dvq = d * var_q

        lsc_v = lsc_r[...]
        dmc = mu_q - muc_r[...]
        sq_c = jnp.sum(dmc * dmc, axis=1, keepdims=True)
        klc = 0.5 * (2.0 * d * (lsc_v - lq)
                     + (dvq + sq_c) * jnp.exp(-2.0 * lsc_v) - d)
        kl_blk = jnp.sum(klc)

        dp3 = mu_q[None] - mupos_v
        dpp = dp3 * dp3
        dq3 = mu_q[None] - muneg_v
        dqq = dq3 * dq3
        sqp = jnp.zeros((bb, wn), f32)
        sqn = jnp.zeros((bb, wn), f32)
        for j in range(wn):
            cj = (lax.broadcasted_iota(jnp.int32, (d, wn), 1) == j).astype(f32)
            sqp = sqp + lax.dot_general(dpp[j], cj, dn2, preferred_element_type=f32)
            sqn = sqn + lax.dot_general(dqq[j], cj, dn2, preferred_element_type=f32)
        lsp_v = lsp_r[...]
        lsn_v = lsn_r[...]
        delta = (d * (lsp_v - lsn_v)
                 + 0.5 * ((dvq + sqp) * jnp.exp(-2.0 * lsp_v)
                          - (dvq + sqn) * jnp.exp(-2.0 * lsn_v)))
        mm_blk = jnp.sum(maskf * jnp.maximum(delta + margin, 0.0))

        @pl.when(ib == 0)
        def _():
            kl_r[...] = jnp.zeros((1, 1), f32)
            mm_r[...] = jnp.zeros((1, 1), f32)

        kl_r[...] += (kl_blk / total_b)[None, None]
        mm_r[...] += (mm_blk / total_b)[None, None]

    full = lambda b: (0, 0)
    out = pl.pallas_call(
        body,
        grid=(nblk,),
        in_specs=[
            pl.BlockSpec((bb, d), lambda b: (b, 0)),
            pl.BlockSpec((bb, d), lambda b: (b, 0)),
            pl.BlockSpec((bb, 1), lambda b: (b, 0)),
            pl.BlockSpec((bb, 1), lambda b: (b, 0)),
            pl.BlockSpec((1, bb), lambda b: (0, b)),
            pl.BlockSpec((1, wn, bb, d), lambda b: (b, 0, 0, 0)),
            pl.BlockSpec((1, wn, bb, d), lambda b: (b, 0, 0, 0)),
            pl.BlockSpec((1, wn, bb, d), lambda b: (b, 0, 0, 0)),
            pl.BlockSpec((bb, wn), lambda b: (b, 0)),
            pl.BlockSpec((bb, wn), lambda b: (b, 0)),
            pl.BlockSpec((h, d), full),
            pl.BlockSpec((h, d), full),
            pl.BlockSpec((1, h), full),
            pl.BlockSpec((d, h), full),
            pl.BlockSpec((1, d), full),
            pl.BlockSpec((1, h), full),
            pl.BlockSpec((1, 1), full),
        ],
        out_specs=[pl.BlockSpec((1, 1), full), pl.BlockSpec((1, 1), full)],
        out_shape=[jax.ShapeDtypeStruct((1, 1), f32),
                   jax.ShapeDtypeStruct((1, 1), f32)],
    )(ce, muc, lsc, ncnt, ncnt.reshape(1, bn), cte3, mupos3, muneg3, lsp, lsn,
      f_w1, f_w2, f_b, u_w, u_b, v_w, v_b)
    return out


def kernel(emb_mu, emb_log_sigma, enc_emb, f_W, f_b, u_W, u_b, v_W, v_b,
           center_ids, context_ids, neg_context_ids, num_contexts):
    bn, wn = context_ids.shape
    v, d = emb_mu.shape
    h = f_W.shape[0]
    margin = 1.0
    ls_flat = emb_log_sigma.reshape(-1)
    ncnt_all = num_contexts.reshape(bn, 1).astype(jnp.int32)

    nsplit = 1
    bs = bn // nsplit
    kl_tot = jnp.zeros((), jnp.float32)
    mm_tot = jnp.zeros((), jnp.float32)
    for s in range(nsplit):
        sl = slice(s * bs, (s + 1) * bs)
        ctx_t = context_ids[sl].T.reshape(-1).astype(jnp.int32)
        neg_t = neg_context_ids[sl].T.reshape(-1).astype(jnp.int32)
        cen = center_ids[sl].astype(jnp.int32)

        bb = 256
        (cte, mupos, muneg, ce, muc, lsp_r, lsn_r, lsc) = _sc_gather_all(
            enc_emb, emb_mu, ls_flat, ctx_t, neg_t, cen, bb)

        kl, mm = _tc_loss(
            ce.reshape(bs, d), muc.reshape(bs, d), lsc.reshape(bs, 1),
            ncnt_all[sl],
            cte.reshape(bs // bb, wn, bb, d), mupos.reshape(bs // bb, wn, bb, d),
            muneg.reshape(bs // bb, wn, bb, d),
            lsp_r.reshape(wn, bs).T, lsn_r.reshape(wn, bs).T,
            f_W[:, :d], f_W[:, d:], f_b.reshape(1, h),
            u_W, u_b.reshape(1, d), v_W, v_b.reshape(1, 1), margin, bn)
        kl_tot = kl_tot + kl[0, 0]
        mm_tot = mm_tot + mm[0, 0]
    return (kl_tot, mm_tot)


# speedup vs baseline: 1.0905x; 1.0160x over previous
import functools

import jax
import jax.numpy as jnp
from jax import lax
from jax.experimental import pallas as pl
from jax.experimental.pallas import tpu as pltpu
from jax.experimental.pallas import tpu_sc as plsc


def _sc_gather_all(enc_emb, emb_mu, ls_flat, ctx_t, neg_t, cen, bb_tc):
    v, d = enc_emb.shape
    bw = ctx_t.shape[0]
    bn = cen.shape[0]
    info = plsc.get_sparse_core_info()
    nw = info.num_cores * info.num_subcores
    ch = 128
    per_w = bw // nw
    n_ch = per_w // ch
    n_pair = n_ch // 2
    cper = bn // nw
    assert per_w % ch == 0 and n_ch % 2 == 0 and bn % nw == 0
    wn_ids = bw // bn
    rpb = bb_tc // ch
    assert bb_tc % ch == 0 and bn % bb_tc == 0 and bn % ch == 0 and cper == ch

    ctx2 = ctx_t.reshape(nw, n_ch, ch)
    neg2 = neg_t.reshape(nw, n_ch, ch)
    cen2 = cen.reshape(nw, 1, cper)
    mesh = plsc.VectorSubcoreMesh(core_axis_name="c", subcore_axis_name="s")
    f32 = jnp.float32

    nb = 4
    scratch = [
        pltpu.VMEM((n_ch, ch), jnp.int32),
        pltpu.VMEM((n_ch, ch), jnp.int32),
        pltpu.VMEM((1, cper), jnp.int32),
    ]
    scratch += [pltpu.VMEM((ch, d), f32) for _ in range(nb)]
    scratch += [
        pltpu.VMEM((n_ch, ch), f32),
        pltpu.VMEM((n_ch, ch), f32),
        pltpu.VMEM((1, cper), f32),
    ]
    scratch += [pltpu.SemaphoreType.DMA] * (2 * nb + 1)

    @functools.partial(
        pl.kernel,
        mesh=mesh,
        compiler_params=pltpu.CompilerParams(use_tc_tiling_on_sc=False),
        out_type=[
            jax.ShapeDtypeStruct((bw // ch, ch, d), f32),
            jax.ShapeDtypeStruct((bw // ch, ch, d), f32),
            jax.ShapeDtypeStruct((bw // ch, ch, d), f32),
            jax.ShapeDtypeStruct((nw, cper, d), f32),
            jax.ShapeDtypeStruct((nw, cper, d), f32),
            jax.ShapeDtypeStruct((nw, n_ch, ch), f32),
            jax.ShapeDtypeStruct((nw, n_ch, ch), f32),
            jax.ShapeDtypeStruct((nw, 1, cper), f32),
        ],
        scratch_types=scratch,
    )
    def sc_kernel(enc_hbm, mu_hbm, ls_hbm, ctx_hbm, neg_hbm, cen_hbm,
                  cte_o, mupos_o, muneg_o, ce_o, muc_o, lsp_o, lsn_o, lsc_o,
                  ctxi, negi, ceni, *rest):
        bufs = list(rest[:nb])
        lspb, lsnb, lscb = rest[nb:nb + 3]
        semg = list(rest[nb + 3:2 * nb + 3])
        semw = list(rest[2 * nb + 3:3 * nb + 3])
        sem2 = rest[3 * nb + 3]
        wid = lax.axis_index("s") * info.num_cores + lax.axis_index("c")

        pltpu.sync_copy(ctx_hbm.at[wid], ctxi)
        pltpu.sync_copy(neg_hbm.at[wid], negi)
        pltpu.sync_copy(cen_hbm.at[wid], ceni)

        @pl.loop(0, n_ch)
        def _ls_issue(c):
            pltpu.async_copy(ls_hbm.at[ctxi.at[c]], lspb.at[c], sem2)
            pltpu.async_copy(ls_hbm.at[negi.at[c]], lsnb.at[c], sem2)

        pltpu.async_copy(ls_hbm.at[ceni.at[0]], lscb.at[0], sem2)

        base = wid * per_w

        def dst(c):
            p = base + c * ch
            j = p // bn
            i = p - j * bn
            blk = i // bb_tc
            r = (i - blk * bb_tc) // ch
            return (blk * wn_ids + j) * rpb + r

        pairs = []
        for tbl, idxs, out in ((enc_hbm, ctxi, cte_o),
                               (mu_hbm, ctxi, mupos_o),
                               (mu_hbm, negi, muneg_o)):
            for c in range(n_ch):
                pairs.append((tbl.at[idxs.at[c]], out.at[dst(c)]))
        pairs.append((enc_hbm.at[ceni.at[0]], ce_o.at[wid]))
        pairs.append((mu_hbm.at[ceni.at[0]], muc_o.at[wid]))
        tot = len(pairs)

        for cc in range(tot + nb - 1):
            if cc < tot:
                k = cc % nb
                if cc >= nb:
                    pltpu.make_async_copy(bufs[k], pairs[cc - nb][1],
                                          semw[k]).wait()
                pltpu.async_copy(pairs[cc][0], bufs[k], semg[k])
            dd = cc - (nb - 1)
            if 0 <= dd < tot:
                k2 = dd % nb
                pltpu.make_async_copy(pairs[dd][0], bufs[k2], semg[k2]).wait()
                pltpu.async_copy(bufs[k2], pairs[dd][1], semw[k2])

        for dd in range(tot - nb, tot):
            k = dd % nb
            pltpu.make_async_copy(bufs[k], pairs[dd][1], semw[k]).wait()

        @pl.loop(0, n_ch)
        def _ls_wait(c):
            pltpu.make_async_copy(ls_hbm.at[ctxi.at[c]], lspb.at[c], sem2).wait()
            pltpu.make_async_copy(ls_hbm.at[negi.at[c]], lsnb.at[c], sem2).wait()

        pltpu.make_async_copy(ls_hbm.at[ceni.at[0]], lscb.at[0], sem2).wait()
        pltpu.sync_copy(lspb, lsp_o.at[wid])
        pltpu.sync_copy(lsnb, lsn_o.at[wid])
        pltpu.sync_copy(lscb, lsc_o.at[wid])

    return sc_kernel(enc_emb, emb_mu, ls_flat, ctx2, neg2, cen2)


def _tc_loss(ce, muc, lsc, ncnt, cte3, mupos3, muneg3, lsp, lsn,
             f_w1, f_w2, f_b, u_w, u_b, v_w, v_b, margin, total_b):
    bn, d = ce.shape
    wn = cte3.shape[1]
    h = f_w1.shape[0]
    bb = cte3.shape[2]
    nblk = bn // bb
    f32 = jnp.float32
    dn = (((1,), (1,)), ((), ()))

    dn2 = (((1,), (0,)), ((), ()))

    def body(ce_r, muc_r, lsc_r, nc_r, ncw_r, cte_r, mupos_r, muneg_r,
             lsp_r, lsn_r, fw1_r, fw2_r, fb_r, uw_r, ub_r, vw_r, vb_r,
             kl_r, mm_r):
        ib = pl.program_id(0)
        a = lax.dot_general(ce_r[...], fw1_r[...], dn, preferred_element_type=f32)
        fb_v = fb_r[...]
        nc_v = nc_r[...]
        iota = lax.broadcasted_iota(jnp.int32, (bb, wn), 1)
        maskf = (iota < nc_v).astype(f32)
        iota_j = lax.broadcasted_iota(jnp.int32, (wn, bb), 0)
        maskw = (iota_j < ncw_r[...]).astype(f32)

        cte_f = cte_r[...].reshape(wn * bb, d)
        mupos_v = mupos_r[...].reshape(wn, bb, d)
        muneg_v = muneg_r[...].reshape(wn, bb, d)
        bj = lax.dot_general(cte_f, fw2_r[...], dn,
                             preferred_element_type=f32).reshape(wn, bb, h)
        hj = jnp.maximum(bj + a[None] + fb_v[None], 0.0)
        h_sum = jnp.sum(hj * maskw[:, :, None], axis=0)
        nvalid = jnp.sum(maskf, axis=1, keepdims=True)
        h_sum = h_sum + (wn - nvalid) * jnp.maximum(fb_v, 0.0)

        mu_q = lax.dot_general(h_sum, uw_r[...], dn, preferred_element_type=f32) + ub_r[...]
        lq = (jnp.sum(h_sum * vw_r[...], axis=1, keepdims=True)
              + vb_r[...])
        var_q = jnp.exp(2.0 * lq)
        dvq = d * var_q

        lsc_v = lsc_r[...]
        dmc = mu_q - muc_r[...]
        sq_c = jnp.sum(dmc * dmc, axis=1, keepdims=True)
        klc = 0.5 * (2.0 * d * (lsc_v - lq)
                     + (dvq + sq_c) * jnp.exp(-2.0 * lsc_v) - d)
        kl_blk = jnp.sum(klc)

        dp3 = mu_q[None] - mupos_v
        dpp = dp3 * dp3
        dq3 = mu_q[None] - muneg_v
        dqq = dq3 * dq3
        sqp = jnp.zeros((bb, wn), f32)
        sqn = jnp.zeros((bb, wn), f32)
        for j in range(wn):
            cj = (lax.broadcasted_iota(jnp.int32, (d, wn), 1) == j).astype(f32)
            sqp = sqp + lax.dot_general(dpp[j], cj, dn2, preferred_element_type=f32)
            sqn = sqn + lax.dot_general(dqq[j], cj, dn2, preferred_element_type=f32)
        lsp_v = lsp_r[...]
        lsn_v = lsn_r[...]
        delta = (d * (lsp_v - lsn_v)
                 + 0.5 * ((dvq + sqp) * jnp.exp(-2.0 * lsp_v)
                          - (dvq + sqn) * jnp.exp(-2.0 * lsn_v)))
        mm_blk = jnp.sum(maskf * jnp.maximum(delta + margin, 0.0))

        @pl.when(ib == 0)
        def _():
            kl_r[...] = jnp.zeros((1, 1), f32)
            mm_r[...] = jnp.zeros((1, 1), f32)

        kl_r[...] += (kl_blk / total_b)[None, None]
        mm_r[...] += (mm_blk / total_b)[None, None]

    full = lambda b: (0, 0)
    out = pl.pallas_call(
        body,
        grid=(nblk,),
        in_specs=[
            pl.BlockSpec((bb, d), lambda b: (b, 0)),
            pl.BlockSpec((bb, d), lambda b: (b, 0)),
            pl.BlockSpec((bb, 1), lambda b: (b, 0)),
            pl.BlockSpec((bb, 1), lambda b: (b, 0)),
            pl.BlockSpec((1, bb), lambda b: (0, b)),
            pl.BlockSpec((1, wn, bb, d), lambda b: (b, 0, 0, 0)),
            pl.BlockSpec((1, wn, bb, d), lambda b: (b, 0, 0, 0)),
            pl.BlockSpec((1, wn, bb, d), lambda b: (b, 0, 0, 0)),
            pl.BlockSpec((bb, wn), lambda b: (b, 0)),
            pl.BlockSpec((bb, wn), lambda b: (b, 0)),
            pl.BlockSpec((h, d), full),
            pl.BlockSpec((h, d), full),
            pl.BlockSpec((1, h), full),
            pl.BlockSpec((d, h), full),
            pl.BlockSpec((1, d), full),
            pl.BlockSpec((1, h), full),
            pl.BlockSpec((1, 1), full),
        ],
        out_specs=[pl.BlockSpec((1, 1), full), pl.BlockSpec((1, 1), full)],
        out_shape=[jax.ShapeDtypeStruct((1, 1), f32),
                   jax.ShapeDtypeStruct((1, 1), f32)],
    )(ce, muc, lsc, ncnt, ncnt.reshape(1, bn), cte3, mupos3, muneg3, lsp, lsn,
      f_w1, f_w2, f_b, u_w, u_b, v_w, v_b)
    return out


def kernel(emb_mu, emb_log_sigma, enc_emb, f_W, f_b, u_W, u_b, v_W, v_b,
           center_ids, context_ids, neg_context_ids, num_contexts):
    bn, wn = context_ids.shape
    v, d = emb_mu.shape
    h = f_W.shape[0]
    margin = 1.0
    ls_flat = emb_log_sigma.reshape(-1)
    ncnt_all = num_contexts.reshape(bn, 1).astype(jnp.int32)

    nsplit = 1
    bs = bn // nsplit
    kl_tot = jnp.zeros((), jnp.float32)
    mm_tot = jnp.zeros((), jnp.float32)
    for s in range(nsplit):
        sl = slice(s * bs, (s + 1) * bs)
        ctx_t = context_ids[sl].T.reshape(-1).astype(jnp.int32)
        neg_t = neg_context_ids[sl].T.reshape(-1).astype(jnp.int32)
        cen = center_ids[sl].astype(jnp.int32)

        bb = 256
        (cte, mupos, muneg, ce, muc, lsp_r, lsn_r, lsc) = _sc_gather_all(
            enc_emb, emb_mu, ls_flat, ctx_t, neg_t, cen, bb)

        kl, mm = _tc_loss(
            ce.reshape(bs, d), muc.reshape(bs, d), lsc.reshape(bs, 1),
            ncnt_all[sl],
            cte.reshape(bs // bb, wn, bb, d), mupos.reshape(bs // bb, wn, bb, d),
            muneg.reshape(bs // bb, wn, bb, d),
            lsp_r.reshape(wn, bs).T, lsn_r.reshape(wn, bs).T,
            f_W[:, :d], f_W[:, d:], f_b.reshape(1, h),
            u_W, u_b.reshape(1, d), v_W, v_b.reshape(1, 1), margin, bn)
        kl_tot = kl_tot + kl[0, 0]
        mm_tot = mm_tot + mm[0, 0]
    return (kl_tot, mm_tot)
